# baseline (speedup 1.0000x reference)
import functools

import jax
import jax.numpy as jnp
from jax import lax
from jax.experimental import pallas as pl
from jax.experimental.pallas import tpu as pltpu
from jax.experimental.pallas import tpu_sc as plsc

D = 1024

PAD_INT, PAD_TEST, PAD_Q, PAD_TAG = 128, 1664, 9472, 1024
OFF_TEST = PAD_INT
OFF_Q = OFF_TEST + PAD_TEST
OFF_TAG = OFF_Q + PAD_Q
R_TOTAL = OFF_TAG + PAD_TAG

BM = 128
NBLK = R_TOTAL // BM
TB1, TB2, TB3 = OFF_TEST // BM, OFF_Q // BM, OFF_TAG // BM

NC_SC, NS_SC = 2, 16
NW = NC_SC * NS_SC
CHUNK = 16


def _proj_body(a_ref, w_ref, o_ref):
    o_ref[...] = jnp.dot(a_ref[...], w_ref[...],
                         preferred_element_type=jnp.float32
                         ).astype(jnp.bfloat16)


def _project_tables(t_cat, w_comb):
    def w_index(i):
        tid = ((i >= TB1).astype(jnp.int32) + (i >= TB2).astype(jnp.int32)
               + (i >= TB3).astype(jnp.int32))
        return (tid, 0)

    return pl.pallas_call(
        _proj_body,
        grid=(NBLK,),
        in_specs=[pl.BlockSpec((BM, D), lambda i: (i, 0)),
                  pl.BlockSpec((D, D), w_index)],
        out_specs=pl.BlockSpec((BM, D), lambda i: (i, 0)),
        out_shape=jax.ShapeDtypeStruct((R_TOTAL, D), jnp.bfloat16),
    )(t_cat, w_comb)


def _make_gather_sum(rows):
    rpw = rows // NW
    nch = rpw // CHUNK
    gc = 4 * CHUNK
    d2 = D // 2
    mesh = plsc.VectorSubcoreMesh(core_axis_name="c", subcore_axis_name="s")
    buf_ty = pltpu.VMEM((gc, d2), jnp.int32)

    @functools.partial(
        pl.kernel,
        mesh=mesh,
        out_type=jax.ShapeDtypeStruct((rows, d2), jnp.int32),
        scratch_types=[pltpu.VMEM((4 * rpw,), jnp.int32),
                       buf_ty, buf_ty,
                       pltpu.SemaphoreType.DMA, pltpu.SemaphoreType.DMA],
    )
    def gather_sum(p_hbm, idx_hbm, out_hbm, idx_v, ga, gb, sem_a, sem_b):
        wid = lax.axis_index("s") * NC_SC + lax.axis_index("c")
        base = wid * rpw
        pltpu.sync_copy(idx_hbm.at[pl.ds(4 * base, 4 * rpw)], idx_v)

        def issue(ci, gbuf, sem):
            @pl.when(ci < nch)
            def _():
                pltpu.async_copy(
                    p_hbm.at[idx_v.at[pl.ds(ci * gc, gc)]], gbuf, sem)

        def wait(gbuf, sem):
            pltpu.make_async_copy(p_hbm.at[pl.ds(0, gc)], gbuf, sem).wait()

        def add_rows(gbuf):
            mh = jnp.int32(-65536)
            rnd = jnp.int32(0x8000)

            def hi(w):
                return lax.bitcast_convert_type(w & mh, jnp.float32)

            def lo(w):
                return lax.bitcast_convert_type(w << 16, jnp.float32)

            def add_row(r, _):
                for k in range(d2 // 16):
                    sl = pl.ds(k * 16, 16)
                    w0 = gbuf[r, sl]
                    w1 = gbuf[CHUNK + r, sl]
                    w2 = gbuf[2 * CHUNK + r, sl]
                    w3 = gbuf[3 * CHUNK + r, sl]
                    sh = (hi(w0) + hi(w1)) + (hi(w2) + hi(w3))
                    sl_ = (lo(w0) + lo(w1)) + (lo(w2) + lo(w3))
                    bh = lax.bitcast_convert_type(sh, jnp.int32) + rnd
                    bl = lax.bitcast_convert_type(sl_, jnp.int32) + rnd
                    gbuf[r, sl] = (bh & mh) | lax.shift_right_logical(
                        bl, 16)
                return 0

            lax.fori_loop(0, CHUNK, add_row, 0)

        def writeback(ci, gbuf):
            pltpu.sync_copy(gbuf.at[pl.ds(0, CHUNK)],
                            out_hbm.at[pl.ds(base + ci * CHUNK, CHUNK)])

        issue(0, ga, sem_a)

        def pair(g, _):
            c0 = 2 * g
            issue(c0 + 1, gb, sem_b)
            wait(ga, sem_a)
            add_rows(ga)
            writeback(c0, ga)
            issue(c0 + 2, ga, sem_a)
            wait(gb, sem_b)
            add_rows(gb)
            writeback(c0 + 1, gb)
            return 0

        lax.fori_loop(0, nch // 2, pair, 0)

    return gather_sum


def _make_ln_body(rb, seq):
    def _ln_body(x_ref, b_ref, g_ref, bb_ref, o_ref):
        x = x_ref[...].astype(jnp.float32) + b_ref[...]
        mu = jnp.mean(x, axis=1, keepdims=True)
        xc = x - mu
        var = jnp.mean(xc * xc, axis=1, keepdims=True)
        y = xc * lax.rsqrt(var + 1e-6) * g_ref[...] + bb_ref[...]
        for j in range(rb):
            o_ref[j] = y[j * seq:(j + 1) * seq, :]
    return _ln_body


def _layernorm(ssum, b, g, bb, bsz, seq):
    rb = 16
    bl = rb * seq
    vec = pl.BlockSpec((1, D), lambda i: (0, 0))
    return pl.pallas_call(
        _make_ln_body(rb, seq),
        grid=(bsz // rb,),
        in_specs=[pl.BlockSpec((bl, D), lambda i: (i, 0)), vec, vec, vec],
        out_specs=pl.BlockSpec((rb, seq, D), lambda i: (i, 0, 0)),
        out_shape=jax.ShapeDtypeStruct((bsz, seq, D), jnp.float32),
    )(ssum, b.reshape(1, D), g.reshape(1, D), bb.reshape(1, D))


def kernel(testId, assessmentItemID, KnowledgeTag, answerCode, mask,
           interaction, emb_interaction, emb_test, emb_question, emb_tag,
           W_comb, b_comb, ln_g, ln_b):
    bsz, seq = interaction.shape
    rows = bsz * seq
    rpw = rows // NW
    nch = rpw // CHUNK

    def padto(x, n):
        return jnp.pad(x, ((0, n - x.shape[0]), (0, 0)))

    t_cat = jnp.concatenate([
        padto(emb_interaction, PAD_INT),
        padto(emb_test, PAD_TEST),
        padto(emb_question, PAD_Q),
        padto(emb_tag, PAD_TAG),
    ], axis=0)

    p_cat = _project_tables(t_cat, W_comb)
    p_i32 = lax.bitcast_convert_type(
        p_cat.reshape(R_TOTAL, D // 2, 2), jnp.int32)

    idx4 = jnp.stack([
        interaction.reshape(rows),
        testId.reshape(rows) + OFF_TEST,
        assessmentItemID.reshape(rows) + OFF_Q,
        KnowledgeTag.reshape(rows) + OFF_TAG,
    ]).astype(jnp.int32)
    idx = (idx4.reshape(4, NW, nch, CHUNK)
           .transpose(1, 2, 0, 3)
           .reshape(4 * rows))

    ssum_i32 = _make_gather_sum(rows)(p_i32, idx)
    ssum = lax.bitcast_convert_type(ssum_i32,
                                    jnp.bfloat16).reshape(rows, D)
    x = _layernorm(ssum, b_comb, ln_g, ln_b, bsz, seq)
    return (x, bsz)


# speedup vs baseline: 3.2501x; 3.2501x over previous
import functools

import jax
import jax.numpy as jnp
from jax import lax
from jax.experimental import pallas as pl
from jax.experimental.pallas import tpu as pltpu
from jax.experimental.pallas import tpu_sc as plsc

D = 1024
BM = 128

NC_SC, NS_SC = 2, 16
NW = NC_SC * NS_SC
CHUNK = 40

TB = 512
K_INT, K_TEST, K_TAG = 8, 1664, 1024


def _proj_body(a_ref, w_ref, o_ref):
    o_ref[...] = jnp.dot(a_ref[...], w_ref[...],
                         preferred_element_type=jnp.float32)


def _project_table(table, w_comb, t):
    v = table.shape[0]
    return pl.pallas_call(
        _proj_body,
        grid=(pl.cdiv(v, BM),),
        in_specs=[pl.BlockSpec((BM, D), lambda i: (i, 0)),
                  pl.BlockSpec((D, D), lambda i: (t, 0))],
        out_specs=pl.BlockSpec((BM, D), lambda i: (i, 0)),
        out_shape=jax.ShapeDtypeStruct((v, D), jnp.float32),
    )(table, w_comb)


def _make_gather(rows):
    rpw = rows // NW
    nch = rpw // CHUNK
    mesh = plsc.VectorSubcoreMesh(core_axis_name="c", subcore_axis_name="s")
    buf_ty = pltpu.VMEM((CHUNK, D), jnp.float32)

    @functools.partial(
        pl.kernel,
        mesh=mesh,
        out_type=jax.ShapeDtypeStruct((rows, D), jnp.float32),
        scratch_types=[pltpu.VMEM((rpw,), jnp.int32), buf_ty, buf_ty,
                       pltpu.SemaphoreType.DMA, pltpu.SemaphoreType.DMA],
    )
    def gather(p_hbm, idx_hbm, out_hbm, idx_v, ga, gb, sem_a, sem_b):
        wid = lax.axis_index("s") * NC_SC + lax.axis_index("c")
        base = wid * rpw
        pltpu.sync_copy(idx_hbm.at[pl.ds(base, rpw)], idx_v)

        def issue(ci, gbuf, sem):
            @pl.when(ci < nch)
            def _():
                pltpu.async_copy(
                    p_hbm.at[idx_v.at[pl.ds(ci * CHUNK, CHUNK)]], gbuf, sem)

        def wait(gbuf, sem):
            pltpu.make_async_copy(p_hbm.at[pl.ds(0, CHUNK)], gbuf,
                                  sem).wait()

        def writeback(ci, gbuf):
            pltpu.sync_copy(gbuf,
                            out_hbm.at[pl.ds(base + ci * CHUNK, CHUNK)])

        issue(0, ga, sem_a)

        def pair(g, _):
            c0 = 2 * g
            issue(c0 + 1, gb, sem_b)
            wait(ga, sem_a)
            writeback(c0, ga)
            issue(c0 + 2, ga, sem_a)
            wait(gb, sem_b)
            writeback(c0 + 1, gb)
            return 0

        lax.fori_loop(0, nch // 2, pair, 0)

    return gather


def _onehot_body(ii_ref, it_ref, ig_ref, p0_ref, p2_ref, p3_ref, o_ref):
    def onehot(iref, k):
        idx = iref[0, 0, :]
        cols = lax.broadcasted_iota(jnp.int32, (TB, k), 1)
        return (cols == idx[:, None]).astype(jnp.bfloat16)

    acc = jnp.dot(onehot(ii_ref, K_INT), p0_ref[...],
                  preferred_element_type=jnp.float32)
    acc += jnp.dot(onehot(it_ref, K_TEST), p2_ref[...],
                   preferred_element_type=jnp.float32)
    acc += jnp.dot(onehot(ig_ref, K_TAG), p3_ref[...],
                   preferred_element_type=jnp.float32)
    o_ref[...] = acc


def _onehot_partial(ii, it, ig, p0, p2, p3, rows):
    nb = rows // TB
    iblk = pl.BlockSpec((1, 1, TB), lambda i: (i, 0, 0))
    return pl.pallas_call(
        _onehot_body,
        grid=(nb,),
        in_specs=[iblk, iblk, iblk,
                  pl.BlockSpec((K_INT, D), lambda i: (0, 0)),
                  pl.BlockSpec((K_TEST, D), lambda i: (0, 0)),
                  pl.BlockSpec((K_TAG, D), lambda i: (0, 0))],
        out_specs=pl.BlockSpec((TB, D), lambda i: (i, 0)),
        out_shape=jax.ShapeDtypeStruct((rows, D), jnp.float32),
    )(ii.reshape(nb, 1, TB), it.reshape(nb, 1, TB), ig.reshape(nb, 1, TB),
      p0, p2, p3)


def _make_ln_body(rb, seq):
    def _ln_body(q_ref, p_ref, b_ref, g_ref, bb_ref, o_ref):
        x = q_ref[...] + p_ref[...] + b_ref[...]
        mu = jnp.mean(x, axis=1, keepdims=True)
        xc = x - mu
        var = jnp.mean(xc * xc, axis=1, keepdims=True)
        y = xc * lax.rsqrt(var + 1e-6) * g_ref[...] + bb_ref[...]
        for j in range(rb):
            o_ref[j] = y[j * seq:(j + 1) * seq, :]
    return _ln_body


def _layernorm(qrows, partial, b, g, bb, bsz, seq):
    rb = 16
    bl = rb * seq
    vec = pl.BlockSpec((1, D), lambda i: (0, 0))
    blk = pl.BlockSpec((bl, D), lambda i: (i, 0))
    return pl.pallas_call(
        _make_ln_body(rb, seq),
        grid=(bsz // rb,),
        in_specs=[blk, blk, vec, vec, vec],
        out_specs=pl.BlockSpec((rb, seq, D), lambda i: (i, 0, 0)),
        out_shape=jax.ShapeDtypeStruct((bsz, seq, D), jnp.float32),
    )(qrows, partial, b.reshape(1, D), g.reshape(1, D), bb.reshape(1, D))


def kernel(testId, assessmentItemID, KnowledgeTag, answerCode, mask,
           interaction, emb_interaction, emb_test, emb_question, emb_tag,
           W_comb, b_comb, ln_g, ln_b):
    bsz, seq = interaction.shape
    rows = bsz * seq

    p_int = _project_table(emb_interaction, W_comb, 0)
    p_test = _project_table(emb_test, W_comb, 1)
    p_q = _project_table(emb_question, W_comb, 2)
    p_tag = _project_table(emb_tag, W_comb, 3)

    q_idx = assessmentItemID.reshape(rows).astype(jnp.int32)
    qrows = _make_gather(rows)(p_q, q_idx)

    def padto(x, n):
        return jnp.pad(x, ((0, n - x.shape[0]), (0, 0)))

    partial = _onehot_partial(
        interaction.reshape(rows).astype(jnp.int32),
        testId.reshape(rows).astype(jnp.int32),
        KnowledgeTag.reshape(rows).astype(jnp.int32),
        padto(p_int.astype(jnp.bfloat16), K_INT),
        padto(p_test.astype(jnp.bfloat16), K_TEST),
        padto(p_tag.astype(jnp.bfloat16), K_TAG),
        rows)

    x = _layernorm(qrows, partial, b_comb, ln_g, ln_b, bsz, seq)
    return (x, bsz)
